# Initial kernel scaffold; baseline (speedup 1.0000x reference)
import jax
import jax.numpy as jnp
from jax.experimental import pallas as pl


def kernel(key_mem, val_mem, query_key, new_key, new_val, idx):
    raise NotImplementedError("write your pallas kernel here")


# speedup vs baseline: 1.1158x; 1.1158x over previous
import functools

import jax
import jax.numpy as jnp
from jax import lax
from jax.experimental import pallas as pl
from jax.experimental.pallas import tpu as pltpu

_B, _CK, _CV, _T, _HW = 2, 64, 512, 20, 900
_THW = _T * _HW
_MP = 18048
_K = 50
_NBLK = 384


def _update_body(idx_ref, mem_ref, new_ref, out_ref):
    t_ids = lax.broadcasted_iota(jnp.int32, mem_ref.shape, 2)
    out_ref[...] = jnp.where(t_ids == idx_ref[0],
                             new_ref[...][:, :, None, :], mem_ref[...])


def _scatter_update(mem, new, idx_arr, c, cb):
    return pl.pallas_call(
        _update_body,
        grid=(_B, c // cb),
        in_specs=[
            pl.BlockSpec(memory_space=pltpu.SMEM),
            pl.BlockSpec((1, cb, _T, _HW), lambda b, j: (b, j, 0, 0)),
            pl.BlockSpec((1, cb, _HW), lambda b, j: (b, j, 0)),
        ],
        out_specs=pl.BlockSpec((1, cb, _T, _HW), lambda b, j: (b, j, 0, 0)),
        out_shape=jax.ShapeDtypeStruct((_B, c, _T, _HW), jnp.float32),
    )(idx_arr, mem, new)


def _aff_body(qkt_ref, km_ref, aff_ref):
    j = pl.program_id(1)
    a = lax.dot_general(
        qkt_ref[0], km_ref[0],
        (((1,), (0,)), ((), ())),
        preferred_element_type=jnp.float32,
    ) * 0.125
    m = j * _NBLK + lax.broadcasted_iota(jnp.int32, a.shape, 1)
    aff_ref[0] = jnp.where(m < _THW, a, -1e30)


def _affinity(qkt, km):
    return pl.pallas_call(
        _aff_body,
        grid=(_B, _MP // _NBLK),
        in_specs=[
            pl.BlockSpec((1, _HW, _CK), lambda b, j: (b, 0, 0)),
            pl.BlockSpec((1, _CK, _NBLK), lambda b, j: (b, 0, j)),
        ],
        out_specs=pl.BlockSpec((1, _HW, _NBLK), lambda b, j: (b, 0, j)),
        out_shape=jax.ShapeDtypeStruct((_B, _HW, _MP), jnp.float32),
    )(qkt, km)


def kernel(key_mem, val_mem, query_key, new_key, new_val, idx):
    idx_arr = jnp.asarray(idx, jnp.int32).reshape(1)
    km4 = key_mem.reshape(_B, _CK, _T, _HW)
    vm4 = val_mem.reshape(_B, _CV, _T, _HW)
    nk = new_key.reshape(_B, _CK, _HW)
    nv = new_val.reshape(_B, _CV, _HW)

    km_out = _scatter_update(km4, nk, idx_arr, _CK, _CK)
    vm_out = _scatter_update(vm4, nv, idx_arr, _CV, 128)

    qkt = query_key.reshape(_B, _CK, _HW).transpose(0, 2, 1)
    aff = _affinity(qkt, km_out.reshape(_B, _CK, _THW))

    topv, topi = lax.top_k(aff, _K)
    w = jax.nn.softmax(topv, axis=-1)
    vt = vm_out.reshape(_B, _CV, _THW).transpose(0, 2, 1)
    g = jnp.take_along_axis(vt[:, None], topi[..., None], axis=2)
    readout = jnp.sum(w[..., None] * g, axis=2)
    readout = readout.transpose(0, 2, 1).reshape(_B, _CV, 30, 30)

    return (readout,
            km_out.reshape(_B, _CK, _T, 30, 30),
            vm_out.reshape(_B, _CV, _T, 30, 30))


# speedup vs baseline: 13.2537x; 11.8784x over previous
import functools

import jax
import jax.numpy as jnp
from jax import lax
from jax.experimental import pallas as pl
from jax.experimental.pallas import tpu as pltpu
from jax.experimental.pallas import tpu_sc as plsc

_B, _CK, _CV, _T, _HW = 2, 64, 512, 20, 900
_THW = _T * _HW
_MP = 18048
_K = 50
_MC = 384
_NC = _MP // _MC
_QB = 225
_QG = _HW // _QB
_NW = 32
_GC = 4
_CW = _GC * _T * _HW
_NCH = _B * _CV // _NW // _GC
_L = 16
_NV16 = (_HW + _L - 1) // _L


def _sc_update_body(km_h, nk_h, vm_h, nv_h, idx_h, kout_h, vout_h,
                    idx_v, buf, nbuf):
    wid = lax.axis_index("s") * 2 + lax.axis_index("c")
    pltpu.sync_copy(idx_h, idx_v)
    idx = idx_v[...][0]
    tail = lax.iota(jnp.int32, _L) < (_HW - (_NV16 - 1) * _L)
    jt = (_NV16 - 1) * _L

    def chunk(mem_h, new_h, out_h, g0):
        pltpu.sync_copy(mem_h.at[pl.ds(g0 * _T * _HW, _CW)],
                        buf.at[pl.ds(0, _CW)])
        pltpu.sync_copy(new_h.at[pl.ds(g0 * _HW, _GC * _HW)],
                        nbuf.at[pl.ds(0, _GC * _HW)])
        for rr in range(_GC):
            base = rr * _T * _HW + idx * _HW
            for j in range(_NV16 - 1):
                buf[pl.ds(base + j * _L, _L)] = nbuf[pl.ds(rr * _HW + j * _L,
                                                           _L)]
            old = buf[pl.ds(base + jt, _L)]
            new = nbuf[pl.ds(rr * _HW + jt, _L)]
            buf[pl.ds(base + jt, _L)] = jnp.where(tail, new, old)
        pltpu.sync_copy(buf.at[pl.ds(0, _CW)],
                        out_h.at[pl.ds(g0 * _T * _HW, _CW)])

    for i in range(_NCH):
        chunk(vm_h, nv_h, vout_h, (wid * _NCH + i) * _GC)
    chunk(km_h, nk_h, kout_h, wid * _GC)


def _sc_update(km, nk, vm, nv, idx_arr):
    mesh = plsc.VectorSubcoreMesh(core_axis_name="c", subcore_axis_name="s")
    return pl.kernel(
        _sc_update_body,
        mesh=mesh,
        out_type=[
            jax.ShapeDtypeStruct((_B * _CK * _T * _HW,), jnp.float32),
            jax.ShapeDtypeStruct((_B * _CV * _T * _HW,), jnp.float32),
        ],
        scratch_types=[
            pltpu.VMEM((_L,), jnp.int32),
            pltpu.VMEM((_CW + _L,), jnp.float32),
            pltpu.VMEM((_GC * _HW + _L,), jnp.float32),
        ],
    )(km, nk, vm, nv, idx_arr)


def _select_body(qkt_ref, km_ref, afft_ref, tau_ref, mx_ref):
    qkt = qkt_ref[0, 0]
    minf = jnp.float32(-1e30)

    def aff_chunk(i, mx):
        kmc = km_ref[0, :, pl.ds(i * _MC, _MC)]
        a = lax.dot_general(qkt, kmc, (((1,), (0,)), ((), ())),
                            preferred_element_type=jnp.float32) * 0.125
        m_ids = i * _MC + lax.broadcasted_iota(jnp.int32, a.shape, 1)
        a = jnp.where(m_ids < _THW, a, minf)
        afft_ref[0, 0, :, pl.ds(i * _MC, _MC)] = a
        return jnp.maximum(mx, jnp.max(a, axis=1, keepdims=True))

    mx = lax.fori_loop(0, _NC, aff_chunk,
                       jnp.full((_QB, 1), minf, jnp.float32))

    def round_body(r, t2):
        cand = t2 + (jnp.int32(1) << (31 - r))
        tf = lax.bitcast_convert_type(
            jnp.where(cand >= 0, cand, cand ^ jnp.int32(0x7FFFFFFF)),
            jnp.float32)

        def cnt_chunk(i, acc):
            ac = afft_ref[0, 0, :, pl.ds(i * _MC, _MC)]
            return acc + (ac >= tf).astype(jnp.int32)

        acc = lax.fori_loop(0, _NC, cnt_chunk,
                            jnp.zeros((_QB, _MC), jnp.int32))
        c = jnp.sum(acc, axis=1, keepdims=True)
        return jnp.where(c >= _K, cand, t2)

    t2 = lax.fori_loop(0, 32, round_body,
                       jnp.full((_QB, 1), jnp.iinfo(jnp.int32).min))
    tau_ref[0, 0] = t2
    mx_ref[0, 0] = mx


def _select(qkt4, km):
    return pl.pallas_call(
        _select_body,
        grid=(_B, _QG),
        in_specs=[
            pl.BlockSpec((1, 1, _QB, _CK), lambda b, q: (b, q, 0, 0)),
            pl.BlockSpec((1, _CK, _MP), lambda b, q: (b, 0, 0)),
        ],
        out_specs=[
            pl.BlockSpec((1, 1, _QB, _MP), lambda b, q: (b, q, 0, 0)),
            pl.BlockSpec((1, 1, _QB, 1), lambda b, q: (b, q, 0, 0)),
            pl.BlockSpec((1, 1, _QB, 1), lambda b, q: (b, q, 0, 0)),
        ],
        out_shape=[
            jax.ShapeDtypeStruct((_B, _QG, _QB, _MP), jnp.float32),
            jax.ShapeDtypeStruct((_B, _QG, _QB, 1), jnp.int32),
            jax.ShapeDtypeStruct((_B, _QG, _QB, 1), jnp.float32),
        ],
    )(qkt4, km)


def _readout_body(val_ref, afft_ref, tau_ref, mx_ref, out_ref,
                  acc_ref, s_ref):
    j = pl.program_id(1)

    @pl.when(j == 0)
    def _():
        acc_ref[...] = jnp.zeros_like(acc_ref)
        s_ref[...] = jnp.zeros_like(s_ref)

    vc = val_ref[0]
    vm_ids = j * _MC + lax.broadcasted_iota(jnp.int32, vc.shape, 1)
    vc = jnp.where(vm_ids < _THW, vc, 0.0)
    ones = jnp.ones((8, _MC), jnp.float32)
    for g in range(_QG):
        afft = afft_ref[0, g]
        tk = tau_ref[0, g]
        tau_f = lax.bitcast_convert_type(
            jnp.where(tk >= 0, tk, tk ^ jnp.int32(0x7FFFFFFF)), jnp.float32)
        mx = mx_ref[0, g]
        w = jnp.where(afft >= tau_f, jnp.exp(afft - mx), 0.0)
        acc_ref[g] += lax.dot_general(vc, w, (((1,), (1,)), ((), ())),
                                      preferred_element_type=jnp.float32)
        s_ref[g] += lax.dot_general(ones, w, (((1,), (1,)), ((), ())),
                                    preferred_element_type=jnp.float32)

    @pl.when(j == _NC - 1)
    def _():
        for g in range(_QG):
            out_ref[0, g] = acc_ref[g] / s_ref[g, 0:1]


def _readout(val, afft, tau_r, mx_r):
    return pl.pallas_call(
        _readout_body,
        grid=(_B, _NC),
        in_specs=[
            pl.BlockSpec((1, _CV, _MC), lambda b, j: (b, 0, j)),
            pl.BlockSpec((1, _QG, _QB, _MC), lambda b, j: (b, 0, 0, j)),
            pl.BlockSpec((1, _QG, _QB, 1), lambda b, j: (b, 0, 0, 0)),
            pl.BlockSpec((1, _QG, _QB, 1), lambda b, j: (b, 0, 0, 0)),
        ],
        out_specs=pl.BlockSpec((1, _QG, _CV, _QB), lambda b, j: (b, 0, 0, 0)),
        out_shape=jax.ShapeDtypeStruct((_B, _QG, _CV, _QB), jnp.float32),
        scratch_shapes=[
            pltpu.VMEM((_QG, _CV, _QB), jnp.float32),
            pltpu.VMEM((_QG, 8, _QB), jnp.float32),
        ],
    )(val, afft, tau_r, mx_r)


def kernel(key_mem, val_mem, query_key, new_key, new_val, idx):
    idx_arr = jnp.full((16,), jnp.asarray(idx, jnp.int32))
    km1 = key_mem.reshape(-1)
    vm1 = val_mem.reshape(-1)
    nk = new_key.reshape(-1)
    nv = new_val.reshape(-1)

    km_out, vm_out = _sc_update(km1, nk, vm1, nv, idx_arr)

    km = km_out.reshape(_B, _CK, _THW)
    qk = query_key.reshape(_B, _CK, _HW)
    qkt4 = qk.transpose(0, 2, 1).reshape(_B, _QG, _QB, _CK)

    afft, tau, mx = _select(qkt4, km)
    ro4 = _readout(vm_out.reshape(_B, _CV, _THW), afft, tau, mx)
    readout = ro4.transpose(0, 2, 1, 3).reshape(_B, _CV, _HW)


    return (readout.reshape(_B, _CV, 30, 30),
            km_out.reshape(_B, _CK, _T, 30, 30),
            vm_out.reshape(_B, _CV, _T, 30, 30))
